# baseline (speedup 1.0000x reference)
import functools

import jax
import jax.numpy as jnp
from jax import lax
from jax.experimental import pallas as pl
from jax.experimental.pallas import tpu as pltpu
from jax.experimental.pallas import tpu_sc as plsc

_FEATURE_DIM = 1000000
_EMBED_DIM = 16
_BATCH = 4096
_N_FIELDS = 26

_NC = 2
_NS = 16
_NW = _NC * _NS
_BPW = _BATCH // _NW
_KPW = _BPW * _N_FIELDS
_CHUNK = 128
_NCHUNK = _KPW // _CHUNK


def _fm_body(table, idxs, vals, out, idx_v, vals_v, rows_v, out_v, sem):
    wid = lax.axis_index("s") * _NC + lax.axis_index("c")

    pltpu.sync_copy(idxs.at[wid], idx_v)
    pltpu.sync_copy(vals.at[wid], vals_v)

    copies = [
        pltpu.async_copy(
            table.at[idx_v.at[j]],
            rows_v.at[pl.ds(j * _CHUNK, _CHUNK), :],
            sem,
        )
        for j in range(_NCHUNK)
    ]
    for c in copies:
        c.wait()

    def bbody(b, carry):
        base = b * _N_FIELDS
        s = jnp.zeros((_EMBED_DIM,), jnp.float32)
        q = jnp.zeros((_EMBED_DIM,), jnp.float32)
        v0 = vals_v[b, pl.ds(0, _EMBED_DIM)]
        v1 = vals_v[b, pl.ds(_EMBED_DIM, _EMBED_DIM)]
        for f in range(_N_FIELDS):
            e = rows_v[base + f, :]
            src = v0 if f < _EMBED_DIM else v1
            lane = f % _EMBED_DIM
            w = lax.gather(
                src,
                jnp.full((_EMBED_DIM, 1), lane, jnp.int32),
                lax.GatherDimensionNumbers(
                    offset_dims=(),
                    collapsed_slice_dims=(0,),
                    start_index_map=(0,),
                ),
                slice_sizes=(1,),
                mode=lax.GatherScatterMode.PROMISE_IN_BOUNDS,
            )
            t = w * e
            s = s + t
            q = q + t * t
        out_v[b, :] = 0.5 * (s * s - q)
        return carry

    lax.fori_loop(0, _BPW, bbody, 0)

    pltpu.sync_copy(out_v, out.at[pl.ds(wid * _BPW, _BPW), :])


@jax.jit
def kernel(feature_embedding, feature_idx, feature_vals):
    idx_r = feature_idx.reshape(_NW, _NCHUNK, _CHUNK)
    vals_r = jnp.pad(
        feature_vals, ((0, 0), (0, 2 * _EMBED_DIM - _N_FIELDS))
    ).reshape(_NW, _BPW, 2 * _EMBED_DIM)

    mesh = plsc.VectorSubcoreMesh(
        core_axis_name="c", subcore_axis_name="s",
        num_cores=_NC, num_subcores=_NS,
    )
    run = pl.kernel(
        _fm_body,
        out_type=jax.ShapeDtypeStruct((_BATCH, _EMBED_DIM), jnp.float32),
        mesh=mesh,
        scratch_types=[
            pltpu.VMEM((_NCHUNK, _CHUNK), jnp.int32),
            pltpu.VMEM((_BPW, 2 * _EMBED_DIM), jnp.float32),
            pltpu.VMEM((_KPW, _EMBED_DIM), jnp.float32),
            pltpu.VMEM((_BPW, _EMBED_DIM), jnp.float32),
            pltpu.SemaphoreType.DMA,
        ],
        compiler_params=pltpu.CompilerParams(use_tc_tiling_on_sc=False),
    )
    return run(feature_embedding, idx_r, vals_r)
